# speedup vs baseline: 1.0000x
import jax
import jax.numpy as jnp
from jax.experimental import pallas as pl


def kernel(pointcloud, params):
    raise NotImplementedError("write your pallas kernel here")


# speedup vs baseline: 5.8792x; 5.8792x over previous
import functools

import jax
import jax.numpy as jnp
import numpy as np
from jax import lax
from jax.experimental import pallas as pl
from jax.experimental.pallas import tpu as pltpu
from jax.experimental.pallas import tpu_sc as plsc

_NPOINTS = [1024, 256, 128, 64]
_RADII = [[0.05, 0.1], [0.1, 0.2], [0.2, 0.4], [0.4, 0.8]]
_K_A, _K_B = 16, 32
_BN_EPS = 1e-5


def _rup(x, m):
    return (x + m - 1) // m * m


def _fps_body(npoint, nch, xyz_ref, out_ref):
    x = xyz_ref[0, 0]
    y = xyz_ref[0, 1]
    z = xyz_ref[0, 2]
    n = nch * 128
    ji = (lax.broadcasted_iota(jnp.int32, (nch, 128), 0) * 128
          + lax.broadcasted_iota(jnp.int32, (nch, 128), 1))

    def body(i, state):
        dists, far = state
        msk = ji == far
        cx = jnp.sum(jnp.where(msk, x, 0.0))
        cy = jnp.sum(jnp.where(msk, y, 0.0))
        cz = jnp.sum(jnp.where(msk, z, 0.0))
        out_ref[0, pl.ds(i, 1), 0:1] = cx.reshape(1, 1)
        out_ref[0, pl.ds(i, 1), 1:2] = cy.reshape(1, 1)
        out_ref[0, pl.ds(i, 1), 2:3] = cz.reshape(1, 1)
        dx = x - cx
        dy = y - cy
        dz = z - cz
        d = dx * dx + dy * dy + dz * dz
        dists = jnp.minimum(dists, d)
        m = jnp.max(dists)
        far = jnp.min(jnp.where(dists == m, ji, n)).astype(jnp.int32)
        return dists, far

    lax.fori_loop(
        0, npoint, body,
        (jnp.full((nch, 128), 1e10, dtype=jnp.float32), jnp.int32(0)))


def _run_fps(xyz_c, npoint):
    b, _, nch, _ = xyz_c.shape
    return pl.pallas_call(
        functools.partial(_fps_body, npoint, nch),
        grid=(b,),
        in_specs=[pl.BlockSpec((1, 3, nch, 128), lambda i: (i, 0, 0, 0))],
        out_specs=pl.BlockSpec((1, npoint, 8), lambda i: (i, 0, 0)),
        out_shape=jax.ShapeDtypeStruct((b, npoint, 8), jnp.float32),
    )(xyz_c)


def _ball_body(n, r2a, r2b, sblk, xyz_ref, ctr_ref, out_ref):
    x = xyz_ref[0, 0:1, :]
    y = xyz_ref[0, 1:2, :]
    z = xyz_ref[0, 2:3, :]
    cx = ctr_ref[0, :, 0:1]
    cy = ctr_ref[0, :, 1:2]
    cz = ctr_ref[0, :, 2:3]
    dx = x - cx
    dy = y - cy
    dz = z - cz
    d2 = dx * dx + dy * dy + dz * dz
    jf = lax.broadcasted_iota(jnp.int32, (1, 1, n), 2).astype(jnp.float32)
    nlog = int(np.ceil(np.log2(n)))

    def select(mask, k_slots):
        m = mask.astype(jnp.float32)
        c = m
        for sh in [1 << t for t in range(nlog)]:
            c = c + jnp.concatenate(
                [jnp.zeros((sblk, sh), jnp.float32), c[:, : n - sh]], axis=1)
        rank = c - m
        cnt = c[:, n - 1:n]
        kk = lax.broadcasted_iota(jnp.int32, (1, k_slots, 1), 1).astype(jnp.float32)
        sel = (rank[:, None, :] == kk) & mask[:, None, :]
        idxs = jnp.sum(jnp.where(sel, jf, 0.0), axis=2)
        first = jnp.where(cnt > 0.0, idxs[:, 0:1], jnp.float32(n - 1))
        kcol = lax.broadcasted_iota(jnp.int32, (sblk, k_slots), 1).astype(jnp.float32)
        return jnp.where(kcol < cnt, idxs, first)

    ia = select(d2 < r2a, _K_A)
    ib = select(d2 < r2b, _K_B)
    pad = jnp.zeros((sblk, 128 - _K_A - _K_B), jnp.float32)
    out_ref[0] = jnp.concatenate([ia, ib, pad], axis=1).astype(jnp.int32)


def _run_ball(xyz_t8, centers, r2a, r2b, sblk=8):
    b, _, n = xyz_t8.shape
    s = centers.shape[1]
    sblk = min(sblk, s)
    return pl.pallas_call(
        functools.partial(_ball_body, n, r2a, r2b, sblk),
        grid=(b, s // sblk),
        in_specs=[
            pl.BlockSpec((1, 8, n), lambda i, j: (i, 0, 0)),
            pl.BlockSpec((1, sblk, 8), lambda i, j: (i, j, 0)),
        ],
        out_specs=pl.BlockSpec((1, sblk, 128), lambda i, j: (i, j, 0)),
        out_shape=jax.ShapeDtypeStruct((b, s, 128), jnp.int32),
    )(xyz_t8, centers)


def _sc_gather(table, idx_flat):
    tot = idx_flat.shape[0]
    dp = table.shape[1]
    nc, ns = 2, 16
    nw = nc * ns
    b_per_w = tot // nw
    chunk = 8
    for c in range(min(b_per_w, 512), 7, -8):
        if b_per_w % c == 0 and c * dp * 4 <= 140_000:
            chunk = c
            break
    assert tot % (8 * nw) == 0 and b_per_w % chunk == 0
    nchunks = b_per_w // chunk
    mesh = plsc.VectorSubcoreMesh(core_axis_name="c", subcore_axis_name="s")

    @functools.partial(
        pl.kernel, mesh=mesh,
        out_type=jax.ShapeDtypeStruct((tot, dp), jnp.float32),
        scratch_types=[
            pltpu.VMEM((chunk,), jnp.int32),
            pltpu.VMEM((chunk, dp), jnp.float32),
            pltpu.SemaphoreType.DMA,
        ],
    )
    def k(table_hbm, idx_hbm, out_hbm, idx_v, rows_v, sem):
        wid = lax.axis_index("s") * nc + lax.axis_index("c")
        base = wid * b_per_w

        def body(ci, _):
            off = base + ci * chunk
            pltpu.sync_copy(idx_hbm.at[pl.ds(off, chunk)], idx_v)
            pltpu.async_copy(table_hbm.at[idx_v], rows_v, sem).wait()
            pltpu.sync_copy(rows_v, out_hbm.at[pl.ds(off, chunk)])
            return 0

        lax.fori_loop(0, nchunks, body, 0)

    return k(table, idx_flat)


def _sa_body(dp, k, sblk, g_ref, ctr_ref, *refs):
    out_ref = refs[-1]
    wrefs = refs[:-1]
    m = sblk * k
    g = g_ref[0, 0]
    ctr = ctr_ref[0, 0]
    cpad = jnp.concatenate(
        [ctr[:, 0:3], jnp.zeros((m, dp - 3), jnp.float32)], axis=1)
    h = g - cpad
    for li in range(len(wrefs) // 2):
        w = wrefs[2 * li][...]
        bb = wrefs[2 * li + 1][...]
        h = jnp.maximum(
            jnp.dot(h, w, preferred_element_type=jnp.float32) + bb, 0.0)
    cout = h.shape[1]
    out_ref[0] = jnp.max(h.reshape(sblk, k, cout), axis=1)


def _run_sa(g4, ctr_rep, weights, k):
    b, nblocks, m, dp = g4.shape
    sblk = m // k
    s = nblocks * sblk
    cout = weights[-1][0].shape[1]
    wargs = []
    in_specs = [
        pl.BlockSpec((1, 1, m, dp), lambda i, j: (i, j, 0, 0)),
        pl.BlockSpec((1, 1, m, 8), lambda i, j: (i, j, 0, 0)),
    ]
    for w, bb in weights:
        wargs += [w, bb]
        in_specs += [
            pl.BlockSpec(w.shape, lambda i, j: (0, 0)),
            pl.BlockSpec(bb.shape, lambda i, j: (0, 0)),
        ]
    return pl.pallas_call(
        functools.partial(_sa_body, dp, k, sblk),
        grid=(b, nblocks),
        in_specs=in_specs,
        out_specs=pl.BlockSpec((1, sblk, cout), lambda i, j: (i, j, 0)),
        out_shape=jax.ShapeDtypeStruct((b, s, cout), jnp.float32),
    )(g4, ctr_rep, *wargs)


def _fp_body(mp, nblk, unk_ref, kn_ref, f_ref, out_ref):
    ux = unk_ref[0, 0:1, :]
    uy = unk_ref[0, 1:2, :]
    uz = unk_ref[0, 2:3, :]
    kx = kn_ref[0, :, 0:1]
    ky = kn_ref[0, :, 1:2]
    kz = kn_ref[0, :, 2:3]
    dx = kx - ux
    dy = ky - uy
    dz = kz - uz
    d2 = dx * dx + dy * dy + dz * dz
    si = lax.broadcasted_iota(jnp.int32, (mp, nblk), 0)
    dists, ams = [], []
    dcur = d2
    for _ in range(3):
        mn = jnp.min(dcur, axis=0, keepdims=True)
        am = jnp.min(jnp.where(dcur == mn, si, mp), axis=0, keepdims=True)
        dists.append(mn)
        ams.append(am)
        dcur = jnp.where(si == am, jnp.float32(1e30), dcur)
    w = [1.0 / (d + 1e-8) for d in dists]
    norm = (w[0] + w[1]) + w[2]
    w = [wt / norm for wt in w]
    a = jnp.zeros((mp, nblk), jnp.float32)
    for t in range(3):
        a = a + jnp.where(si == ams[t], w[t], 0.0)
    out_ref[0] = jnp.dot(f_ref[0], a, preferred_element_type=jnp.float32)


def _run_fp_interp(unk_t8, kn, feats, nblk):
    b, _, n = unk_t8.shape
    mp = kn.shape[1]
    cp = feats.shape[1]
    nblk = min(nblk, n)
    return pl.pallas_call(
        functools.partial(_fp_body, mp, nblk),
        grid=(b, n // nblk),
        in_specs=[
            pl.BlockSpec((1, 8, nblk), lambda i, j: (i, 0, j)),
            pl.BlockSpec((1, mp, 8), lambda i, j: (i, 0, 0)),
            pl.BlockSpec((1, cp, mp), lambda i, j: (i, 0, 0)),
        ],
        out_specs=pl.BlockSpec((1, cp, nblk), lambda i, j: (i, 0, j)),
        out_shape=jax.ShapeDtypeStruct((b, cp, n), jnp.float32),
    )(unk_t8, kn, feats)


def _chain_body(maxpool_n, x_ref, *refs):
    out_ref = refs[-1]
    wrefs = refs[:-1]
    h = x_ref[0]
    for li in range(len(wrefs) // 2):
        w = wrefs[2 * li][...]
        bb = wrefs[2 * li + 1][...]
        h = jnp.maximum(
            jnp.dot(w, h, preferred_element_type=jnp.float32) + bb, 0.0)
    if maxpool_n:
        out_ref[0] = jnp.max(h[:, :maxpool_n], axis=1, keepdims=True)
    else:
        out_ref[0] = h


def _run_chain(x, weights, nblk=512, maxpool_n=0):
    b, cinp, n = x.shape
    cout = weights[-1][0].shape[0]
    nblk = min(nblk, n)
    wargs = []
    in_specs = [pl.BlockSpec((1, cinp, nblk), lambda i, j: (i, 0, j))]
    for w, bb in weights:
        wargs += [w, bb]
        in_specs += [
            pl.BlockSpec(w.shape, lambda i, j: (0, 0)),
            pl.BlockSpec(bb.shape, lambda i, j: (0, 0)),
        ]
    if maxpool_n:
        out_specs = pl.BlockSpec((1, cout, 1), lambda i, j: (i, 0, 0))
        out_shape = jax.ShapeDtypeStruct((b, cout, 1), jnp.float32)
    else:
        out_specs = pl.BlockSpec((1, cout, nblk), lambda i, j: (i, 0, j))
        out_shape = jax.ShapeDtypeStruct((b, cout, n), jnp.float32)
    return pl.pallas_call(
        functools.partial(_chain_body, maxpool_n),
        grid=(b, n // nblk),
        in_specs=in_specs,
        out_specs=out_specs,
        out_shape=out_shape,
    )(x, *wargs)


def _fold(layer):
    scale = layer["gamma"] / np.sqrt(1.0 + _BN_EPS)
    return layer["W"] * scale[:, None], layer["beta"]


def _prep_chain_weights(layers, cin):
    out = []
    cin_p = _rup(max(cin, 8), 8)
    for lyr in layers:
        w, bvec = _fold(lyr)
        cout, cw = w.shape
        cout_p = _rup(max(cout, 8), 8)
        wp = jnp.zeros((cout_p, cin_p), jnp.float32).at[:cout, :cw].set(w)
        bp = jnp.zeros((cout_p, 1), jnp.float32).at[:cout, 0].set(bvec)
        out.append((wp, bp))
        cin_p = cout_p
    return out


def _prep_sa_weights(layers, cin_p):
    out = []
    for lyr in layers:
        w, bvec = _fold(lyr)
        cout, cw = w.shape
        cout_p = _rup(max(cout, 128), 128)
        wp = jnp.zeros((cin_p, cout_p), jnp.float32).at[:cw, :cout].set(w.T)
        bp = jnp.zeros((1, cout_p), jnp.float32).at[0, :cout].set(bvec)
        out.append((wp, bp))
        cin_p = cout_p
    return out


def kernel(pointcloud, params):
    b, n0, _ = pointcloud.shape
    t0 = jnp.transpose(pointcloud, (0, 2, 1))
    xc = t0.reshape(b, 3, n0 // 128, 128)
    t8 = jnp.concatenate([t0, jnp.zeros((b, 5, n0), jnp.float32)], axis=1)

    feats0 = _run_chain(t8, _prep_chain_weights([params["fc_in"]], 8),
                        nblk=1024)

    l_xyz_t8 = [t8]
    l_centers = [None]
    l_feat = [feats0]

    for li in range(4):
        s = _NPOINTS[li]
        r_a, r_b = _RADII[li]
        centers = _run_fps(xc, s)
        idx = _run_ball(t8, centers, r_a * r_a, r_b * r_b)
        feats = l_feat[li]
        c = feats.shape[1]
        npts = t8.shape[2]
        d = 3 + c
        dp = _rup(d, 128)
        p = jnp.concatenate(
            [jnp.transpose(t8[:, 0:3, :], (0, 2, 1)),
             jnp.transpose(feats, (0, 2, 1)),
             jnp.zeros((b, npts, dp - d), jnp.float32)], axis=2)
        ktot = _K_A + _K_B
        idx_off = (idx[:, :, :ktot]
                   + (jnp.arange(b, dtype=jnp.int32) * npts)[:, None, None])
        rows = _sc_gather(p.reshape(b * npts, dp),
                          idx_off.reshape(b * s * ktot))
        rows = rows.reshape(b, s, ktot, dp)
        outs = []
        for sc, (k, lo) in enumerate([(_K_A, 0), (_K_B, _K_A)]):
            sblk = min(16, s)
            nblocks = s // sblk
            m = sblk * k
            g4 = rows[:, :, lo:lo + k, :].reshape(b, nblocks, m, dp)
            ctr_rep = jnp.repeat(centers, k, axis=1).reshape(b, nblocks, m, 8)
            wlist = _prep_sa_weights(params["sa"][li][sc], dp)
            pooled = _run_sa(g4, ctr_rep, wlist, k)
            cout = params["sa"][li][sc][-1]["W"].shape[0]
            outs.append(jnp.transpose(pooled[:, :, :cout], (0, 2, 1)))
        l_feat.append(jnp.concatenate(outs, axis=1))
        nxt = jnp.transpose(centers[:, :, 0:3], (0, 2, 1))
        t8 = jnp.concatenate([nxt, jnp.zeros((b, 5, s), jnp.float32)], axis=1)
        xc = nxt.reshape(b, 3, s // 128, 128) if s >= 128 \
            else nxt.reshape(b, 3, 1, s)
        l_xyz_t8.append(t8)
        l_centers.append(centers)

    for fp_i, unk_i, kn_i in [(-1, 3, 4), (-2, 2, 3), (-3, 1, 2), (-4, 0, 1)]:
        unk_t8 = l_xyz_t8[unk_i]
        n_unk = unk_t8.shape[2]
        kn = l_centers[kn_i]
        m = kn.shape[1]
        mp = _rup(m, 128)
        kf = l_feat[kn_i]
        c = kf.shape[1]
        if mp != m:
            kn = jnp.concatenate(
                [kn, jnp.full((b, mp - m, 8), 1e6, jnp.float32)], axis=1)
            kf = jnp.concatenate(
                [kf, jnp.zeros((b, c, mp - m), jnp.float32)], axis=2)
        interp = _run_fp_interp(unk_t8, kn, kf, nblk=min(n_unk, 1024))
        x = jnp.concatenate([interp, l_feat[unk_i]], axis=1)
        cin = x.shape[1]
        wlist = _prep_chain_weights(params["fp"][fp_i], cin)
        cin_p = wlist[0][0].shape[1]
        if cin_p != cin:
            x = jnp.concatenate(
                [x, jnp.zeros((b, cin_p - cin, n_unk), jnp.float32)], axis=1)
        l_feat[unk_i] = _run_chain(x, wlist, nblk=min(n_unk, 512))

    feat_pt = _run_chain(l_feat[0], _prep_chain_weights([params["fc_pt"]], 128),
                         nblk=1024)
    feat_pt = jnp.transpose(feat_pt, (0, 2, 1))

    g_in = l_feat[4]
    n_g = g_in.shape[2]
    g_pad = jnp.concatenate(
        [g_in, jnp.zeros((b, g_in.shape[1], 128 - n_g), jnp.float32)], axis=2)
    feat_g = _run_chain(g_pad, _prep_chain_weights([params["fc_g"]], 1024),
                        nblk=128, maxpool_n=n_g)
    return feat_g[:, :, 0], feat_pt


# speedup vs baseline: 10.7078x; 1.8213x over previous
import functools

import jax
import jax.numpy as jnp
import numpy as np
from jax import lax
from jax.experimental import pallas as pl
from jax.experimental.pallas import tpu as pltpu
from jax.experimental.pallas import tpu_sc as plsc

_NPOINTS = [1024, 256, 128, 64]
_RADII = [[0.05, 0.1], [0.1, 0.2], [0.2, 0.4], [0.4, 0.8]]
_K_A, _K_B = 16, 32
_BN_EPS = 1e-5


def _rup(x, m):
    return (x + m - 1) // m * m


def _fps_body(npoint, n, b, xyz_ref, out_ref):
    x = xyz_ref[0]
    y = xyz_ref[1]
    z = xyz_ref[2]
    ji = lax.broadcasted_iota(jnp.int32, (b, n), 1)
    lane = lax.broadcasted_iota(jnp.int32, (b, 128), 1)
    tile = min(npoint, 128)

    def body(i, st, base):
        dists, far, tx, ty, tz = st
        msk = ji == far
        cx = jnp.sum(jnp.where(msk, x, 0.0), axis=1, keepdims=True)
        cy = jnp.sum(jnp.where(msk, y, 0.0), axis=1, keepdims=True)
        cz = jnp.sum(jnp.where(msk, z, 0.0), axis=1, keepdims=True)
        lm = lane == (i - base)
        tx = jnp.where(lm, cx, tx)
        ty = jnp.where(lm, cy, ty)
        tz = jnp.where(lm, cz, tz)
        dx = x - cx
        dy = y - cy
        dz = z - cz
        d = dx * dx + dy * dy + dz * dz
        dists = jnp.minimum(dists, d)
        m = jnp.max(dists, axis=1, keepdims=True)
        far = jnp.min(jnp.where(dists == m, ji, n), axis=1, keepdims=True)
        return dists, far, tx, ty, tz

    dists = jnp.full((b, n), 1e10, dtype=jnp.float32)
    far = jnp.zeros((b, 1), dtype=jnp.int32)
    z128 = jnp.zeros((b, 128), dtype=jnp.float32)
    for o in range(npoint // tile):
        dists, far, tx, ty, tz = lax.fori_loop(
            o * tile, (o + 1) * tile,
            functools.partial(body, base=o * tile),
            (dists, far, z128, z128, z128))
        out_ref[0, :, o * tile:o * tile + tile] = tx[:, :tile]
        out_ref[1, :, o * tile:o * tile + tile] = ty[:, :tile]
        out_ref[2, :, o * tile:o * tile + tile] = tz[:, :tile]


def _run_fps(xyz3, npoint):
    _, b, n = xyz3.shape
    out = pl.pallas_call(
        functools.partial(_fps_body, npoint, n, b),
        in_specs=[pl.BlockSpec((3, b, n), lambda: (0, 0, 0))],
        out_specs=pl.BlockSpec((3, b, npoint), lambda: (0, 0, 0)),
        out_shape=jax.ShapeDtypeStruct((3, b, npoint), jnp.float32),
    )(xyz3)
    ctr = jnp.transpose(out, (1, 2, 0))
    return jnp.concatenate(
        [ctr, jnp.zeros((b, npoint, 5), jnp.float32)], axis=2)


def _ball_body(n, r2a, r2b, sblk, xyz_ref, ctr_ref, out_ref):
    x = xyz_ref[0, 0:1, :]
    y = xyz_ref[0, 1:2, :]
    z = xyz_ref[0, 2:3, :]
    cx = ctr_ref[0, :, 0:1]
    cy = ctr_ref[0, :, 1:2]
    cz = ctr_ref[0, :, 2:3]
    dx = x - cx
    dy = y - cy
    dz = z - cz
    d2 = dx * dx + dy * dy + dz * dz
    jf = lax.broadcasted_iota(jnp.int32, (1, 1, n), 2).astype(jnp.float32)
    nlog = int(np.ceil(np.log2(n)))

    def select(mask, k_slots):
        m = mask.astype(jnp.float32)
        c = m
        for sh in [1 << t for t in range(nlog)]:
            c = c + jnp.concatenate(
                [jnp.zeros((sblk, sh), jnp.float32), c[:, : n - sh]], axis=1)
        rank = c - m
        cnt = c[:, n - 1:n]
        kk = lax.broadcasted_iota(jnp.int32, (1, k_slots, 1), 1).astype(jnp.float32)
        sel = (rank[:, None, :] == kk) & mask[:, None, :]
        idxs = jnp.sum(jnp.where(sel, jf, 0.0), axis=2)
        first = jnp.where(cnt > 0.0, idxs[:, 0:1], jnp.float32(n - 1))
        kcol = lax.broadcasted_iota(jnp.int32, (sblk, k_slots), 1).astype(jnp.float32)
        return jnp.where(kcol < cnt, idxs, first)

    ia = select(d2 < r2a, _K_A)
    ib = select(d2 < r2b, _K_B)
    pad = jnp.zeros((sblk, 128 - _K_A - _K_B), jnp.float32)
    out_ref[0] = jnp.concatenate([ia, ib, pad], axis=1).astype(jnp.int32)


def _run_ball(xyz_t8, centers, r2a, r2b, sblk=8):
    b, _, n = xyz_t8.shape
    s = centers.shape[1]
    sblk = min(sblk, s)
    return pl.pallas_call(
        functools.partial(_ball_body, n, r2a, r2b, sblk),
        grid=(b, s // sblk),
        in_specs=[
            pl.BlockSpec((1, 8, n), lambda i, j: (i, 0, 0)),
            pl.BlockSpec((1, sblk, 8), lambda i, j: (i, j, 0)),
        ],
        out_specs=pl.BlockSpec((1, sblk, 128), lambda i, j: (i, j, 0)),
        out_shape=jax.ShapeDtypeStruct((b, s, 128), jnp.int32),
    )(xyz_t8, centers)


def _sc_gather(table, idx_flat):
    tot = idx_flat.shape[0]
    dp = table.shape[1]
    nc, ns = 2, 16
    nw = nc * ns
    b_per_w = tot // nw
    chunk = 8
    for c in range(min(b_per_w, 512), 7, -8):
        if b_per_w % c == 0 and c * dp * 4 <= 140_000:
            chunk = c
            break
    assert tot % (8 * nw) == 0 and b_per_w % chunk == 0
    nchunks = b_per_w // chunk
    mesh = plsc.VectorSubcoreMesh(core_axis_name="c", subcore_axis_name="s")

    @functools.partial(
        pl.kernel, mesh=mesh,
        out_type=jax.ShapeDtypeStruct((tot, dp), jnp.float32),
        scratch_types=[
            pltpu.VMEM((chunk,), jnp.int32),
            pltpu.VMEM((chunk, dp), jnp.float32),
            pltpu.SemaphoreType.DMA,
        ],
    )
    def k(table_hbm, idx_hbm, out_hbm, idx_v, rows_v, sem):
        wid = lax.axis_index("s") * nc + lax.axis_index("c")
        base = wid * b_per_w

        def body(ci, _):
            off = base + ci * chunk
            pltpu.sync_copy(idx_hbm.at[pl.ds(off, chunk)], idx_v)
            pltpu.async_copy(table_hbm.at[idx_v], rows_v, sem).wait()
            pltpu.sync_copy(rows_v, out_hbm.at[pl.ds(off, chunk)])
            return 0

        lax.fori_loop(0, nchunks, body, 0)

    return k(table, idx_flat)


def _sa_body(dp, k, sblk, g_ref, ctr_ref, *refs):
    out_ref = refs[-1]
    wrefs = refs[:-1]
    m = sblk * k
    g = g_ref[0, 0]
    ctr = ctr_ref[0, 0]
    cpad = jnp.concatenate(
        [ctr[:, 0:3], jnp.zeros((m, dp - 3), jnp.float32)], axis=1)
    h = g - cpad
    for li in range(len(wrefs) // 2):
        w = wrefs[2 * li][...]
        bb = wrefs[2 * li + 1][...]
        h = jnp.maximum(
            jnp.dot(h, w, preferred_element_type=jnp.float32) + bb, 0.0)
    cout = h.shape[1]
    out_ref[0] = jnp.max(h.reshape(sblk, k, cout), axis=1)


def _run_sa(g4, ctr_rep, weights, k):
    b, nblocks, m, dp = g4.shape
    sblk = m // k
    s = nblocks * sblk
    cout = weights[-1][0].shape[1]
    wargs = []
    in_specs = [
        pl.BlockSpec((1, 1, m, dp), lambda i, j: (i, j, 0, 0)),
        pl.BlockSpec((1, 1, m, 8), lambda i, j: (i, j, 0, 0)),
    ]
    for w, bb in weights:
        wargs += [w, bb]
        in_specs += [
            pl.BlockSpec(w.shape, lambda i, j: (0, 0)),
            pl.BlockSpec(bb.shape, lambda i, j: (0, 0)),
        ]
    return pl.pallas_call(
        functools.partial(_sa_body, dp, k, sblk),
        grid=(b, nblocks),
        in_specs=in_specs,
        out_specs=pl.BlockSpec((1, sblk, cout), lambda i, j: (i, j, 0)),
        out_shape=jax.ShapeDtypeStruct((b, s, cout), jnp.float32),
    )(g4, ctr_rep, *wargs)


def _fp_body(mp, nblk, unk_ref, kn_ref, f_ref, out_ref):
    ux = unk_ref[0, 0:1, :]
    uy = unk_ref[0, 1:2, :]
    uz = unk_ref[0, 2:3, :]
    kx = kn_ref[0, :, 0:1]
    ky = kn_ref[0, :, 1:2]
    kz = kn_ref[0, :, 2:3]
    dx = kx - ux
    dy = ky - uy
    dz = kz - uz
    d2 = dx * dx + dy * dy + dz * dz
    si = lax.broadcasted_iota(jnp.int32, (mp, nblk), 0)
    dists, ams = [], []
    dcur = d2
    for _ in range(3):
        mn = jnp.min(dcur, axis=0, keepdims=True)
        am = jnp.min(jnp.where(dcur == mn, si, mp), axis=0, keepdims=True)
        dists.append(mn)
        ams.append(am)
        dcur = jnp.where(si == am, jnp.float32(1e30), dcur)
    w = [1.0 / (d + 1e-8) for d in dists]
    norm = (w[0] + w[1]) + w[2]
    w = [wt / norm for wt in w]
    a = jnp.zeros((mp, nblk), jnp.float32)
    for t in range(3):
        a = a + jnp.where(si == ams[t], w[t], 0.0)
    out_ref[0] = jnp.dot(f_ref[0], a, preferred_element_type=jnp.float32)


def _run_fp_interp(unk_t8, kn, feats, nblk):
    b, _, n = unk_t8.shape
    mp = kn.shape[1]
    cp = feats.shape[1]
    nblk = min(nblk, n)
    return pl.pallas_call(
        functools.partial(_fp_body, mp, nblk),
        grid=(b, n // nblk),
        in_specs=[
            pl.BlockSpec((1, 8, nblk), lambda i, j: (i, 0, j)),
            pl.BlockSpec((1, mp, 8), lambda i, j: (i, 0, 0)),
            pl.BlockSpec((1, cp, mp), lambda i, j: (i, 0, 0)),
        ],
        out_specs=pl.BlockSpec((1, cp, nblk), lambda i, j: (i, 0, j)),
        out_shape=jax.ShapeDtypeStruct((b, cp, n), jnp.float32),
    )(unk_t8, kn, feats)


def _chain_body(maxpool_n, x_ref, *refs):
    out_ref = refs[-1]
    wrefs = refs[:-1]
    h = x_ref[0]
    for li in range(len(wrefs) // 2):
        w = wrefs[2 * li][...]
        bb = wrefs[2 * li + 1][...]
        h = jnp.maximum(
            jnp.dot(w, h, preferred_element_type=jnp.float32) + bb, 0.0)
    if maxpool_n:
        out_ref[0] = jnp.max(h[:, :maxpool_n], axis=1, keepdims=True)
    else:
        out_ref[0] = h


def _run_chain(x, weights, nblk=512, maxpool_n=0):
    b, cinp, n = x.shape
    cout = weights[-1][0].shape[0]
    nblk = min(nblk, n)
    wargs = []
    in_specs = [pl.BlockSpec((1, cinp, nblk), lambda i, j: (i, 0, j))]
    for w, bb in weights:
        wargs += [w, bb]
        in_specs += [
            pl.BlockSpec(w.shape, lambda i, j: (0, 0)),
            pl.BlockSpec(bb.shape, lambda i, j: (0, 0)),
        ]
    if maxpool_n:
        out_specs = pl.BlockSpec((1, cout, 1), lambda i, j: (i, 0, 0))
        out_shape = jax.ShapeDtypeStruct((b, cout, 1), jnp.float32)
    else:
        out_specs = pl.BlockSpec((1, cout, nblk), lambda i, j: (i, 0, j))
        out_shape = jax.ShapeDtypeStruct((b, cout, n), jnp.float32)
    return pl.pallas_call(
        functools.partial(_chain_body, maxpool_n),
        grid=(b, n // nblk),
        in_specs=in_specs,
        out_specs=out_specs,
        out_shape=out_shape,
    )(x, *wargs)


def _fold(layer):
    scale = layer["gamma"] / np.sqrt(1.0 + _BN_EPS)
    return layer["W"] * scale[:, None], layer["beta"]


def _prep_chain_weights(layers, cin):
    out = []
    cin_p = _rup(max(cin, 8), 8)
    for lyr in layers:
        w, bvec = _fold(lyr)
        cout, cw = w.shape
        cout_p = _rup(max(cout, 8), 8)
        wp = jnp.zeros((cout_p, cin_p), jnp.float32).at[:cout, :cw].set(w)
        bp = jnp.zeros((cout_p, 1), jnp.float32).at[:cout, 0].set(bvec)
        out.append((wp, bp))
        cin_p = cout_p
    return out


def _prep_sa_weights(layers, cin_p):
    out = []
    for lyr in layers:
        w, bvec = _fold(lyr)
        cout, cw = w.shape
        cout_p = _rup(max(cout, 128), 128)
        wp = jnp.zeros((cin_p, cout_p), jnp.float32).at[:cw, :cout].set(w.T)
        bp = jnp.zeros((1, cout_p), jnp.float32).at[0, :cout].set(bvec)
        out.append((wp, bp))
        cin_p = cout_p
    return out


def kernel(pointcloud, params):
    b, n0, _ = pointcloud.shape
    t0 = jnp.transpose(pointcloud, (0, 2, 1))
    xc = jnp.transpose(pointcloud, (2, 0, 1))
    t8 = jnp.concatenate([t0, jnp.zeros((b, 5, n0), jnp.float32)], axis=1)

    feats0 = _run_chain(t8, _prep_chain_weights([params["fc_in"]], 8),
                        nblk=1024)

    l_xyz_t8 = [t8]
    l_centers = [None]
    l_feat = [feats0]

    for li in range(4):
        s = _NPOINTS[li]
        r_a, r_b = _RADII[li]
        centers = _run_fps(xc, s)
        idx = _run_ball(t8, centers, r_a * r_a, r_b * r_b)
        feats = l_feat[li]
        c = feats.shape[1]
        npts = t8.shape[2]
        d = 3 + c
        dp = _rup(d, 128)
        p = jnp.concatenate(
            [jnp.transpose(t8[:, 0:3, :], (0, 2, 1)),
             jnp.transpose(feats, (0, 2, 1)),
             jnp.zeros((b, npts, dp - d), jnp.float32)], axis=2)
        ktot = _K_A + _K_B
        idx_off = (idx[:, :, :ktot]
                   + (jnp.arange(b, dtype=jnp.int32) * npts)[:, None, None])
        rows = _sc_gather(p.reshape(b * npts, dp),
                          idx_off.reshape(b * s * ktot))
        rows = rows.reshape(b, s, ktot, dp)
        outs = []
        for sc, (k, lo) in enumerate([(_K_A, 0), (_K_B, _K_A)]):
            sblk = min(16, s)
            nblocks = s // sblk
            m = sblk * k
            g4 = rows[:, :, lo:lo + k, :].reshape(b, nblocks, m, dp)
            ctr_rep = jnp.repeat(centers, k, axis=1).reshape(b, nblocks, m, 8)
            wlist = _prep_sa_weights(params["sa"][li][sc], dp)
            pooled = _run_sa(g4, ctr_rep, wlist, k)
            cout = params["sa"][li][sc][-1]["W"].shape[0]
            outs.append(jnp.transpose(pooled[:, :, :cout], (0, 2, 1)))
        l_feat.append(jnp.concatenate(outs, axis=1))
        nxt = jnp.transpose(centers[:, :, 0:3], (0, 2, 1))
        t8 = jnp.concatenate([nxt, jnp.zeros((b, 5, s), jnp.float32)], axis=1)
        xc = jnp.transpose(centers[:, :, 0:3], (2, 0, 1))
        l_xyz_t8.append(t8)
        l_centers.append(centers)

    for fp_i, unk_i, kn_i in [(-1, 3, 4), (-2, 2, 3), (-3, 1, 2), (-4, 0, 1)]:
        unk_t8 = l_xyz_t8[unk_i]
        n_unk = unk_t8.shape[2]
        kn = l_centers[kn_i]
        m = kn.shape[1]
        mp = _rup(m, 128)
        kf = l_feat[kn_i]
        c = kf.shape[1]
        if mp != m:
            kn = jnp.concatenate(
                [kn, jnp.full((b, mp - m, 8), 1e6, jnp.float32)], axis=1)
            kf = jnp.concatenate(
                [kf, jnp.zeros((b, c, mp - m), jnp.float32)], axis=2)
        interp = _run_fp_interp(unk_t8, kn, kf, nblk=min(n_unk, 1024))
        x = jnp.concatenate([interp, l_feat[unk_i]], axis=1)
        cin = x.shape[1]
        wlist = _prep_chain_weights(params["fp"][fp_i], cin)
        cin_p = wlist[0][0].shape[1]
        if cin_p != cin:
            x = jnp.concatenate(
                [x, jnp.zeros((b, cin_p - cin, n_unk), jnp.float32)], axis=1)
        l_feat[unk_i] = _run_chain(x, wlist, nblk=min(n_unk, 512))

    feat_pt = _run_chain(l_feat[0], _prep_chain_weights([params["fc_pt"]], 128),
                         nblk=1024)
    feat_pt = jnp.transpose(feat_pt, (0, 2, 1))

    g_in = l_feat[4]
    n_g = g_in.shape[2]
    g_pad = jnp.concatenate(
        [g_in, jnp.zeros((b, g_in.shape[1], 128 - n_g), jnp.float32)], axis=2)
    feat_g = _run_chain(g_pad, _prep_chain_weights([params["fc_g"]], 1024),
                        nblk=128, maxpool_n=n_g)
    return feat_g[:, :, 0], feat_pt


# speedup vs baseline: 11.7463x; 1.0970x over previous
import functools

import jax
import jax.numpy as jnp
import numpy as np
from jax import lax
from jax.experimental import pallas as pl
from jax.experimental.pallas import tpu as pltpu
from jax.experimental.pallas import tpu_sc as plsc

_NPOINTS = [1024, 256, 128, 64]
_RADII = [[0.05, 0.1], [0.1, 0.2], [0.2, 0.4], [0.4, 0.8]]
_K_A, _K_B = 16, 32
_BN_EPS = 1e-5


def _rup(x, m):
    return (x + m - 1) // m * m


def _fps_body(npoint, n, b, xyz_ref, out_ref):
    x = xyz_ref[0]
    y = xyz_ref[1]
    z = xyz_ref[2]
    ji = lax.broadcasted_iota(jnp.int32, (b, n), 1)
    lane = lax.broadcasted_iota(jnp.int32, (b, 128), 1)
    tile = min(npoint, 128)

    def body(i, st, base):
        dists, far, tx, ty, tz = st
        msk = ji == far
        cx = jnp.sum(jnp.where(msk, x, 0.0), axis=1, keepdims=True)
        cy = jnp.sum(jnp.where(msk, y, 0.0), axis=1, keepdims=True)
        cz = jnp.sum(jnp.where(msk, z, 0.0), axis=1, keepdims=True)
        lm = lane == (i - base)
        tx = jnp.where(lm, cx, tx)
        ty = jnp.where(lm, cy, ty)
        tz = jnp.where(lm, cz, tz)
        dx = x - cx
        dy = y - cy
        dz = z - cz
        d = dx * dx + dy * dy + dz * dz
        dists = jnp.minimum(dists, d)
        m = jnp.max(dists, axis=1, keepdims=True)
        far = jnp.min(jnp.where(dists == m, ji, n), axis=1, keepdims=True)
        return dists, far, tx, ty, tz

    dists = jnp.full((b, n), 1e10, dtype=jnp.float32)
    far = jnp.zeros((b, 1), dtype=jnp.int32)
    z128 = jnp.zeros((b, 128), dtype=jnp.float32)
    for o in range(npoint // tile):
        dists, far, tx, ty, tz = lax.fori_loop(
            o * tile, (o + 1) * tile,
            functools.partial(body, base=o * tile),
            (dists, far, z128, z128, z128))
        out_ref[0, :, o * tile:o * tile + tile] = tx[:, :tile]
        out_ref[1, :, o * tile:o * tile + tile] = ty[:, :tile]
        out_ref[2, :, o * tile:o * tile + tile] = tz[:, :tile]


def _run_fps(xyz3, npoint):
    _, b, n = xyz3.shape
    out = pl.pallas_call(
        functools.partial(_fps_body, npoint, n, b),
        in_specs=[pl.BlockSpec((3, b, n), lambda: (0, 0, 0))],
        out_specs=pl.BlockSpec((3, b, npoint), lambda: (0, 0, 0)),
        out_shape=jax.ShapeDtypeStruct((3, b, npoint), jnp.float32),
    )(xyz3)
    ctr = jnp.transpose(out, (1, 2, 0))
    return jnp.concatenate(
        [ctr, jnp.zeros((b, npoint, 5), jnp.float32)], axis=2)


def _ball_body(n, r2a, r2b, sblk, xyz_ref, ctr_ref, out_ref):
    x = xyz_ref[0, 0:1, :]
    y = xyz_ref[0, 1:2, :]
    z = xyz_ref[0, 2:3, :]
    cx = ctr_ref[0, :, 0:1]
    cy = ctr_ref[0, :, 1:2]
    cz = ctr_ref[0, :, 2:3]
    dx = x - cx
    dy = y - cy
    dz = z - cz
    d2 = dx * dx + dy * dy + dz * dz
    jf = lax.broadcasted_iota(jnp.int32, (1, 1, n), 2).astype(jnp.float32)
    nlog = int(np.ceil(np.log2(n)))

    ma = (d2 < r2a).astype(jnp.int32)
    mb = (d2 < r2b).astype(jnp.int32)
    c = ma + (mb << 13)
    for sh in [1 << t for t in range(nlog)]:
        c = c + jnp.concatenate(
            [jnp.zeros((sblk, sh), jnp.int32), c[:, : n - sh]], axis=1)
    ca = c & 8191
    cb = c >> 13
    rva = jnp.where(ma == 1, ca - ma, -1)
    rvb = jnp.where(mb == 1, cb - mb, -1)

    def select(rankv, cnt, k_slots):
        kk = lax.broadcasted_iota(jnp.int32, (1, k_slots, 1), 1)
        idxs = jnp.sum(
            jnp.where(rankv[:, None, :] == kk, jf, 0.0), axis=2)
        first = jnp.where(cnt > 0, idxs[:, 0:1], jnp.float32(n - 1))
        kcol = lax.broadcasted_iota(jnp.int32, (sblk, k_slots), 1)
        return jnp.where(kcol < cnt, idxs, first)

    ia = select(rva, ca[:, n - 1:n], _K_A)
    ib = select(rvb, cb[:, n - 1:n], _K_B)
    pad = jnp.zeros((sblk, 128 - _K_A - _K_B), jnp.float32)
    out_ref[0] = jnp.concatenate([ia, ib, pad], axis=1).astype(jnp.int32)


def _run_ball(xyz_t8, centers, r2a, r2b, sblk=8):
    b, _, n = xyz_t8.shape
    s = centers.shape[1]
    sblk = min(sblk, s)
    return pl.pallas_call(
        functools.partial(_ball_body, n, r2a, r2b, sblk),
        grid=(b, s // sblk),
        in_specs=[
            pl.BlockSpec((1, 8, n), lambda i, j: (i, 0, 0)),
            pl.BlockSpec((1, sblk, 8), lambda i, j: (i, j, 0)),
        ],
        out_specs=pl.BlockSpec((1, sblk, 128), lambda i, j: (i, j, 0)),
        out_shape=jax.ShapeDtypeStruct((b, s, 128), jnp.int32),
    )(xyz_t8, centers)


def _sc_gather(table, idx_flat):
    tot = idx_flat.shape[0]
    dp = table.shape[1]
    nc, ns = 2, 16
    nw = nc * ns
    b_per_w = tot // nw
    chunk = 8
    for c in range(min(b_per_w, 512), 7, -8):
        if b_per_w % c == 0 and c * dp * 4 <= 140_000:
            chunk = c
            break
    assert tot % (8 * nw) == 0 and b_per_w % chunk == 0
    nchunks = b_per_w // chunk
    mesh = plsc.VectorSubcoreMesh(core_axis_name="c", subcore_axis_name="s")

    @functools.partial(
        pl.kernel, mesh=mesh,
        out_type=jax.ShapeDtypeStruct((tot, dp), jnp.float32),
        scratch_types=[
            pltpu.VMEM((chunk,), jnp.int32),
            pltpu.VMEM((chunk, dp), jnp.float32),
            pltpu.SemaphoreType.DMA,
        ],
    )
    def k(table_hbm, idx_hbm, out_hbm, idx_v, rows_v, sem):
        wid = lax.axis_index("s") * nc + lax.axis_index("c")
        base = wid * b_per_w

        def body(ci, _):
            off = base + ci * chunk
            pltpu.sync_copy(idx_hbm.at[pl.ds(off, chunk)], idx_v)
            pltpu.async_copy(table_hbm.at[idx_v], rows_v, sem).wait()
            pltpu.sync_copy(rows_v, out_hbm.at[pl.ds(off, chunk)])
            return 0

        lax.fori_loop(0, nchunks, body, 0)

    return k(table, idx_flat)


def _sa_body(dp, k, sblk, g_ref, ctr_ref, *refs):
    out_ref = refs[-1]
    wrefs = refs[:-1]
    m = sblk * k
    g = g_ref[0, 0]
    ctr = ctr_ref[0, 0]
    cpad = jnp.concatenate(
        [ctr[:, 0:3], jnp.zeros((m, dp - 3), jnp.float32)], axis=1)
    h = g - cpad
    for li in range(len(wrefs) // 2):
        w = wrefs[2 * li][...]
        bb = wrefs[2 * li + 1][...]
        h = jnp.maximum(
            jnp.dot(h, w, preferred_element_type=jnp.float32) + bb, 0.0)
    cout = h.shape[1]
    out_ref[0] = jnp.max(h.reshape(sblk, k, cout), axis=1)


def _run_sa(g4, ctr_rep, weights, k):
    b, nblocks, m, dp = g4.shape
    sblk = m // k
    s = nblocks * sblk
    cout = weights[-1][0].shape[1]
    wargs = []
    in_specs = [
        pl.BlockSpec((1, 1, m, dp), lambda i, j: (i, j, 0, 0)),
        pl.BlockSpec((1, 1, m, 8), lambda i, j: (i, j, 0, 0)),
    ]
    for w, bb in weights:
        wargs += [w, bb]
        in_specs += [
            pl.BlockSpec(w.shape, lambda i, j: (0, 0)),
            pl.BlockSpec(bb.shape, lambda i, j: (0, 0)),
        ]
    return pl.pallas_call(
        functools.partial(_sa_body, dp, k, sblk),
        grid=(b, nblocks),
        in_specs=in_specs,
        out_specs=pl.BlockSpec((1, sblk, cout), lambda i, j: (i, j, 0)),
        out_shape=jax.ShapeDtypeStruct((b, s, cout), jnp.float32),
    )(g4, ctr_rep, *wargs)


def _fp_body(mp, nblk, unk_ref, kn_ref, f_ref, out_ref):
    ux = unk_ref[0, 0:1, :]
    uy = unk_ref[0, 1:2, :]
    uz = unk_ref[0, 2:3, :]
    kx = kn_ref[0, :, 0:1]
    ky = kn_ref[0, :, 1:2]
    kz = kn_ref[0, :, 2:3]
    dx = kx - ux
    dy = ky - uy
    dz = kz - uz
    d2 = dx * dx + dy * dy + dz * dz
    si = lax.broadcasted_iota(jnp.int32, (mp, nblk), 0)
    dists, ams = [], []
    dcur = d2
    for _ in range(3):
        mn = jnp.min(dcur, axis=0, keepdims=True)
        am = jnp.min(jnp.where(dcur == mn, si, mp), axis=0, keepdims=True)
        dists.append(mn)
        ams.append(am)
        dcur = jnp.where(si == am, jnp.float32(1e30), dcur)
    w = [1.0 / (d + 1e-8) for d in dists]
    norm = (w[0] + w[1]) + w[2]
    w = [wt / norm for wt in w]
    a = jnp.zeros((mp, nblk), jnp.float32)
    for t in range(3):
        a = a + jnp.where(si == ams[t], w[t], 0.0)
    out_ref[0] = jnp.dot(f_ref[0], a, preferred_element_type=jnp.float32)


def _run_fp_interp(unk_t8, kn, feats, nblk):
    b, _, n = unk_t8.shape
    mp = kn.shape[1]
    cp = feats.shape[1]
    nblk = min(nblk, n)
    return pl.pallas_call(
        functools.partial(_fp_body, mp, nblk),
        grid=(b, n // nblk),
        in_specs=[
            pl.BlockSpec((1, 8, nblk), lambda i, j: (i, 0, j)),
            pl.BlockSpec((1, mp, 8), lambda i, j: (i, 0, 0)),
            pl.BlockSpec((1, cp, mp), lambda i, j: (i, 0, 0)),
        ],
        out_specs=pl.BlockSpec((1, cp, nblk), lambda i, j: (i, 0, j)),
        out_shape=jax.ShapeDtypeStruct((b, cp, n), jnp.float32),
    )(unk_t8, kn, feats)


def _chain_body(maxpool_n, x_ref, *refs):
    out_ref = refs[-1]
    wrefs = refs[:-1]
    h = x_ref[0]
    for li in range(len(wrefs) // 2):
        w = wrefs[2 * li][...]
        bb = wrefs[2 * li + 1][...]
        h = jnp.maximum(
            jnp.dot(w, h, preferred_element_type=jnp.float32) + bb, 0.0)
    if maxpool_n:
        out_ref[0] = jnp.max(h[:, :maxpool_n], axis=1, keepdims=True)
    else:
        out_ref[0] = h


def _run_chain(x, weights, nblk=512, maxpool_n=0):
    b, cinp, n = x.shape
    cout = weights[-1][0].shape[0]
    nblk = min(nblk, n)
    wargs = []
    in_specs = [pl.BlockSpec((1, cinp, nblk), lambda i, j: (i, 0, j))]
    for w, bb in weights:
        wargs += [w, bb]
        in_specs += [
            pl.BlockSpec(w.shape, lambda i, j: (0, 0)),
            pl.BlockSpec(bb.shape, lambda i, j: (0, 0)),
        ]
    if maxpool_n:
        out_specs = pl.BlockSpec((1, cout, 1), lambda i, j: (i, 0, 0))
        out_shape = jax.ShapeDtypeStruct((b, cout, 1), jnp.float32)
    else:
        out_specs = pl.BlockSpec((1, cout, nblk), lambda i, j: (i, 0, j))
        out_shape = jax.ShapeDtypeStruct((b, cout, n), jnp.float32)
    return pl.pallas_call(
        functools.partial(_chain_body, maxpool_n),
        grid=(b, n // nblk),
        in_specs=in_specs,
        out_specs=out_specs,
        out_shape=out_shape,
    )(x, *wargs)


def _fold(layer):
    scale = layer["gamma"] / np.sqrt(1.0 + _BN_EPS)
    return layer["W"] * scale[:, None], layer["beta"]


def _prep_chain_weights(layers, cin):
    out = []
    cin_p = _rup(max(cin, 8), 8)
    for lyr in layers:
        w, bvec = _fold(lyr)
        cout, cw = w.shape
        cout_p = _rup(max(cout, 8), 8)
        wp = jnp.zeros((cout_p, cin_p), jnp.float32).at[:cout, :cw].set(w)
        bp = jnp.zeros((cout_p, 1), jnp.float32).at[:cout, 0].set(bvec)
        out.append((wp, bp))
        cin_p = cout_p
    return out


def _prep_sa_weights(layers, cin_p):
    out = []
    for lyr in layers:
        w, bvec = _fold(lyr)
        cout, cw = w.shape
        cout_p = _rup(max(cout, 128), 128)
        wp = jnp.zeros((cin_p, cout_p), jnp.float32).at[:cw, :cout].set(w.T)
        bp = jnp.zeros((1, cout_p), jnp.float32).at[0, :cout].set(bvec)
        out.append((wp, bp))
        cin_p = cout_p
    return out


def kernel(pointcloud, params):
    b, n0, _ = pointcloud.shape
    t0 = jnp.transpose(pointcloud, (0, 2, 1))
    xc = jnp.transpose(pointcloud, (2, 0, 1))
    t8 = jnp.concatenate([t0, jnp.zeros((b, 5, n0), jnp.float32)], axis=1)

    feats0 = _run_chain(t8, _prep_chain_weights([params["fc_in"]], 8),
                        nblk=1024)

    l_xyz_t8 = [t8]
    l_centers = [None]
    l_feat = [feats0]

    for li in range(4):
        s = _NPOINTS[li]
        r_a, r_b = _RADII[li]
        centers = _run_fps(xc, s)
        idx = _run_ball(t8, centers, r_a * r_a, r_b * r_b)
        feats = l_feat[li]
        c = feats.shape[1]
        npts = t8.shape[2]
        d = 3 + c
        dp = _rup(d, 128)
        p = jnp.concatenate(
            [jnp.transpose(t8[:, 0:3, :], (0, 2, 1)),
             jnp.transpose(feats, (0, 2, 1)),
             jnp.zeros((b, npts, dp - d), jnp.float32)], axis=2)
        ktot = _K_A + _K_B
        idx_off = (idx[:, :, :ktot]
                   + (jnp.arange(b, dtype=jnp.int32) * npts)[:, None, None])
        rows = _sc_gather(p.reshape(b * npts, dp),
                          idx_off.reshape(b * s * ktot))
        rows = rows.reshape(b, s, ktot, dp)
        outs = []
        for sc, (k, lo) in enumerate([(_K_A, 0), (_K_B, _K_A)]):
            sblk = min(16, s)
            nblocks = s // sblk
            m = sblk * k
            g4 = rows[:, :, lo:lo + k, :].reshape(b, nblocks, m, dp)
            ctr_rep = jnp.repeat(centers, k, axis=1).reshape(b, nblocks, m, 8)
            wlist = _prep_sa_weights(params["sa"][li][sc], dp)
            pooled = _run_sa(g4, ctr_rep, wlist, k)
            cout = params["sa"][li][sc][-1]["W"].shape[0]
            outs.append(jnp.transpose(pooled[:, :, :cout], (0, 2, 1)))
        l_feat.append(jnp.concatenate(outs, axis=1))
        nxt = jnp.transpose(centers[:, :, 0:3], (0, 2, 1))
        t8 = jnp.concatenate([nxt, jnp.zeros((b, 5, s), jnp.float32)], axis=1)
        xc = jnp.transpose(centers[:, :, 0:3], (2, 0, 1))
        l_xyz_t8.append(t8)
        l_centers.append(centers)

    for fp_i, unk_i, kn_i in [(-1, 3, 4), (-2, 2, 3), (-3, 1, 2), (-4, 0, 1)]:
        unk_t8 = l_xyz_t8[unk_i]
        n_unk = unk_t8.shape[2]
        kn = l_centers[kn_i]
        m = kn.shape[1]
        mp = _rup(m, 128)
        kf = l_feat[kn_i]
        c = kf.shape[1]
        if mp != m:
            kn = jnp.concatenate(
                [kn, jnp.full((b, mp - m, 8), 1e6, jnp.float32)], axis=1)
            kf = jnp.concatenate(
                [kf, jnp.zeros((b, c, mp - m), jnp.float32)], axis=2)
        interp = _run_fp_interp(unk_t8, kn, kf, nblk=min(n_unk, 1024))
        x = jnp.concatenate([interp, l_feat[unk_i]], axis=1)
        cin = x.shape[1]
        wlist = _prep_chain_weights(params["fp"][fp_i], cin)
        cin_p = wlist[0][0].shape[1]
        if cin_p != cin:
            x = jnp.concatenate(
                [x, jnp.zeros((b, cin_p - cin, n_unk), jnp.float32)], axis=1)
        l_feat[unk_i] = _run_chain(x, wlist, nblk=min(n_unk, 512))

    feat_pt = _run_chain(l_feat[0], _prep_chain_weights([params["fc_pt"]], 128),
                         nblk=1024)
    feat_pt = jnp.transpose(feat_pt, (0, 2, 1))

    g_in = l_feat[4]
    n_g = g_in.shape[2]
    g_pad = jnp.concatenate(
        [g_in, jnp.zeros((b, g_in.shape[1], 128 - n_g), jnp.float32)], axis=2)
    feat_g = _run_chain(g_pad, _prep_chain_weights([params["fc_g"]], 1024),
                        nblk=128, maxpool_n=n_g)
    return feat_g[:, :, 0], feat_pt
